# baseline (speedup 1.0000x reference)
import functools

import jax
import jax.numpy as jnp
from jax import lax
from jax.experimental import pallas as pl
from jax.experimental.pallas import tpu as pltpu
from jax.experimental.pallas import tpu_sc as plsc

B = 16384
S = 200
D = 128
NUM_CLASSES = 2

NW = 32
BPW = B // NW
IB = 32
CH = BPW // IB
PAIRS = CH // 2
NE = 4
NBUF = 2 * NE
GA = 128
GB = S - GA
GW = 100
NCH = D // 16


def _sc_gather_pool(emb_table, ids):
    mesh = plsc.VectorSubcoreMesh(core_axis_name="c", subcore_axis_name="s")

    @functools.partial(
        pl.kernel,
        out_type=jax.ShapeDtypeStruct((B, D), jnp.float32),
        mesh=mesh,
        scratch_types=(
            [pltpu.VMEM((IB, GA), jnp.int32),
             pltpu.VMEM((IB, GB), jnp.int32),
             pltpu.VMEM((IB, GA), jnp.int32),
             pltpu.VMEM((IB, GB), jnp.int32)]
            + [pltpu.VMEM((GA, D), jnp.float32)
               if k % 2 == 0 else
               pltpu.VMEM((GB, D), jnp.float32)
               for k in range(NBUF)]
            + [pltpu.VMEM((IB, D), jnp.float32),
               pltpu.VMEM((IB, D), jnp.float32)]
            + [pltpu.SemaphoreType.DMA for _ in range(NBUF + 3)]
        ),
    )
    def pool_kernel(table_hbm, ids_hbm, out_hbm,
                    idxa0, idxb0, idxa1, idxb1, *rest):
        idxa = (idxa0, idxa1)
        idxb = (idxb0, idxb1)
        rows = rest[:NBUF]
        pooled = rest[NBUF:NBUF + 2]
        sems = rest[NBUF + 2:NBUF + 2 + NBUF]
        idx_sem = rest[NBUF + 2 + NBUF]
        osem = rest[NBUF + 3 + NBUF:NBUF + 5 + NBUF]
        wid = lax.axis_index("s") * 2 + lax.axis_index("c")
        wbase = wid * BPW

        def stage(chunk, par):
            gb = wbase + chunk * IB
            pltpu.async_copy(ids_hbm.at[pl.ds(gb, IB), pl.ds(0, GA)],
                             idxa[par], idx_sem)
            pltpu.async_copy(ids_hbm.at[pl.ds(gb, IB), pl.ds(GA, GB)],
                             idxb[par], idx_sem)

        def stage_wait():
            pltpu.make_async_copy(ids_hbm.at[pl.ds(0, IB), pl.ds(0, GA)],
                                  idxa[0], idx_sem).wait()
            pltpu.make_async_copy(ids_hbm.at[pl.ds(0, IB), pl.ds(GA, GB)],
                                  idxb[0], idx_sem).wait()

        def flush_wait(par):
            pltpu.make_async_copy(pooled[par], out_hbm.at[pl.ds(0, IB)],
                                  osem[par]).wait()

        def fire(bl, e, par):
            pltpu.async_copy(table_hbm.at[idxa[par].at[bl]],
                             rows[2 * e], sems[2 * e])
            pltpu.async_copy(table_hbm.at[idxb[par].at[bl]],
                             rows[2 * e + 1], sems[2 * e + 1])

        def drain(k):
            n = GA if k % 2 == 0 else GB
            pltpu.make_async_copy(table_hbm.at[pl.ds(0, n)], rows[k],
                                  sems[k]).wait()

        def accum(bl, e, par):
            UNROLL = 8

            def part(rbuf, n, acc0):
                def body(i, acc):
                    s0 = i * UNROLL
                    for u in range(UNROLL):
                        acc = tuple(acc[c] + rbuf[s0 + u, pl.ds(c * 16, 16)]
                                    for c in range(NCH))
                    return acc

                return lax.fori_loop(0, n // UNROLL, body, acc0)

            acc = tuple(jnp.zeros((16,), jnp.float32) for _ in range(NCH))
            drain(2 * e)
            acc = part(rows[2 * e], GA, acc)
            drain(2 * e + 1)
            acc = part(rows[2 * e + 1], GB, acc)
            for c in range(NCH):
                pooled[par][bl, pl.ds(c * 16, 16)] = acc[c]

        stage(0, 0)
        stage_wait()
        for e in range(NE):
            fire(e, e, 0)
        stage(1, 1)

        @pl.loop(0, PAIRS)
        def _(p):
            for par in range(2):
                chunk = 2 * p + par
                gbase = wbase + chunk * IB

                @pl.when(chunk >= 2)
                def _():
                    flush_wait(par)

                @pl.loop(0, IB - NE, step=NE)
                def _(j):
                    for e in range(NE):
                        accum(j + e, e, par)
                        fire(j + e + NE, e, par)

                not_last = chunk < CH - 1

                @pl.when(not_last)
                def _():
                    stage_wait()

                for e in range(NE):
                    accum(IB - NE + e, e, par)

                    @pl.when(not_last)
                    def _(e=e):
                        fire(e, e, 1 - par)

                @pl.when(chunk < CH - 2)
                def _():
                    stage(chunk + 2, par)

                pltpu.async_copy(pooled[par], out_hbm.at[pl.ds(gbase, IB)],
                                 osem[par])

        for par in range(2):
            flush_wait(par)

    return pool_kernel(emb_table, ids)


def _tc_head(pooled_sum, fc_w, fc_b2):
    def body(p_ref, w_ref, b_ref, o_ref):
        p = p_ref[...]
        w = w_ref[...]
        o_ref[...] = (
            lax.dot_general(p, w, (((1,), (1,)), ((), ())),
                            preferred_element_type=jnp.float32)
            * (1.0 / S)
            + b_ref[...]
        )

    blk = 8192
    return pl.pallas_call(
        body,
        grid=(B // blk,),
        in_specs=[
            pl.BlockSpec((blk, D), lambda i: (i, 0)),
            pl.BlockSpec((NUM_CLASSES, D), lambda i: (0, 0)),
            pl.BlockSpec((1, NUM_CLASSES), lambda i: (0, 0)),
        ],
        out_specs=pl.BlockSpec((blk, NUM_CLASSES), lambda i: (i, 0)),
        out_shape=jax.ShapeDtypeStruct((B, NUM_CLASSES), jnp.float32),
    )(pooled_sum, fc_w, fc_b2)


def kernel(input_ids, attention_mask, emb_table, fc_w, fc_b):
    del attention_mask
    pooled_sum = _sc_gather_pool(emb_table, input_ids.astype(jnp.int32))
    return _tc_head(pooled_sum, fc_w, fc_b.reshape(1, NUM_CLASSES))


# speedup vs baseline: 1.0020x; 1.0020x over previous
import functools

import jax
import jax.numpy as jnp
from jax import lax
from jax.experimental import pallas as pl
from jax.experimental.pallas import tpu as pltpu
from jax.experimental.pallas import tpu_sc as plsc

B = 16384
S = 200
D = 128
NUM_CLASSES = 2

NW = 32
BPW = B // NW
IB = 32
CH = BPW // IB
PAIRS = CH // 2
NE = 4
NBUF = 2 * NE
GA = 128
GB = S - GA
GW = 100
NCH = D // 16


def _sc_gather_pool(emb_table, ids):
    mesh = plsc.VectorSubcoreMesh(core_axis_name="c", subcore_axis_name="s")

    @functools.partial(
        pl.kernel,
        out_type=jax.ShapeDtypeStruct((B, D), jnp.float32),
        mesh=mesh,
        scratch_types=(
            [pltpu.VMEM((IB, GA), jnp.int32),
             pltpu.VMEM((IB, GB), jnp.int32),
             pltpu.VMEM((IB, GA), jnp.int32),
             pltpu.VMEM((IB, GB), jnp.int32)]
            + [pltpu.VMEM((GA, D), jnp.float32)
               if k % 2 == 0 else
               pltpu.VMEM((GB, D), jnp.float32)
               for k in range(NBUF)]
            + [pltpu.VMEM((IB, D), jnp.float32),
               pltpu.VMEM((IB, D), jnp.float32)]
            + [pltpu.SemaphoreType.DMA for _ in range(NBUF + 1)]
        ),
    )
    def pool_kernel(table_hbm, ids_hbm, out_hbm,
                    idxa0, idxb0, idxa1, idxb1, *rest):
        idxa = (idxa0, idxa1)
        idxb = (idxb0, idxb1)
        rows = rest[:NBUF]
        pooled = rest[NBUF:NBUF + 2]
        sems = rest[NBUF + 2:NBUF + 2 + NBUF]
        idx_sem = rest[NBUF + 2 + NBUF]
        wid = lax.axis_index("s") * 2 + lax.axis_index("c")
        wbase = wid * BPW

        def stage(chunk, par):
            gb = wbase + chunk * IB
            pltpu.async_copy(ids_hbm.at[pl.ds(gb, IB), pl.ds(0, GA)],
                             idxa[par], idx_sem)
            pltpu.async_copy(ids_hbm.at[pl.ds(gb, IB), pl.ds(GA, GB)],
                             idxb[par], idx_sem)

        def stage_wait():
            pltpu.make_async_copy(ids_hbm.at[pl.ds(0, IB), pl.ds(0, GA)],
                                  idxa[0], idx_sem).wait()
            pltpu.make_async_copy(ids_hbm.at[pl.ds(0, IB), pl.ds(GA, GB)],
                                  idxb[0], idx_sem).wait()

        def fire(bl, e, par):
            pltpu.async_copy(table_hbm.at[idxa[par].at[bl]],
                             rows[2 * e], sems[2 * e])
            pltpu.async_copy(table_hbm.at[idxb[par].at[bl]],
                             rows[2 * e + 1], sems[2 * e + 1])

        def drain(k):
            n = GA if k % 2 == 0 else GB
            pltpu.make_async_copy(table_hbm.at[pl.ds(0, n)], rows[k],
                                  sems[k]).wait()

        def accum(bl, e, par):
            UNROLL = 8

            def part(rbuf, n, acc0):
                def body(i, acc):
                    s0 = i * UNROLL
                    for u in range(UNROLL):
                        acc = tuple(acc[c] + rbuf[s0 + u, pl.ds(c * 16, 16)]
                                    for c in range(NCH))
                    return acc

                return lax.fori_loop(0, n // UNROLL, body, acc0)

            acc = tuple(jnp.zeros((16,), jnp.float32) for _ in range(NCH))
            drain(2 * e)
            acc = part(rows[2 * e], GA, acc)
            drain(2 * e + 1)
            acc = part(rows[2 * e + 1], GB, acc)
            for c in range(NCH):
                pooled[par][bl, pl.ds(c * 16, 16)] = acc[c]

        stage(0, 0)
        stage_wait()
        for e in range(NE):
            fire(e, e, 0)
        stage(1, 1)

        @pl.loop(0, PAIRS)
        def _(p):
            for par in range(2):
                chunk = 2 * p + par
                gbase = wbase + chunk * IB

                @pl.loop(0, IB - NE, step=NE)
                def _(j):
                    for e in range(NE):
                        accum(j + e, e, par)
                        fire(j + e + NE, e, par)

                not_last = chunk < CH - 1

                @pl.when(not_last)
                def _():
                    stage_wait()

                for e in range(NE):
                    accum(IB - NE + e, e, par)

                    @pl.when(not_last)
                    def _(e=e):
                        fire(e, e, 1 - par)

                @pl.when(chunk < CH - 2)
                def _():
                    stage(chunk + 2, par)

                pltpu.sync_copy(pooled[par], out_hbm.at[pl.ds(gbase, IB)])

    return pool_kernel(emb_table, ids)


def _tc_head(pooled_sum, fc_w, fc_b2):
    def body(p_ref, w_ref, b_ref, o_ref):
        p = p_ref[...]
        w = w_ref[...]
        o_ref[...] = (
            lax.dot_general(p, w, (((1,), (1,)), ((), ())),
                            preferred_element_type=jnp.float32)
            * (1.0 / S)
            + b_ref[...]
        )

    blk = 8192
    return pl.pallas_call(
        body,
        grid=(B // blk,),
        in_specs=[
            pl.BlockSpec((blk, D), lambda i: (i, 0)),
            pl.BlockSpec((NUM_CLASSES, D), lambda i: (0, 0)),
            pl.BlockSpec((1, NUM_CLASSES), lambda i: (0, 0)),
        ],
        out_specs=pl.BlockSpec((blk, NUM_CLASSES), lambda i: (i, 0)),
        out_shape=jax.ShapeDtypeStruct((B, NUM_CLASSES), jnp.float32),
    )(pooled_sum, fc_w, fc_b2)


def kernel(input_ids, attention_mask, emb_table, fc_w, fc_b):
    del attention_mask
    pooled_sum = _sc_gather_pool(emb_table, input_ids.astype(jnp.int32))
    return _tc_head(pooled_sum, fc_w, fc_b.reshape(1, NUM_CLASSES))
